# baseline (speedup 1.0000x reference)
import jax
import jax.numpy as jnp
from jax.experimental import pallas as pl
from jax.experimental.pallas import tpu as pltpu
from jax.experimental.pallas import tpu_sc as plsc

_B, _C, _H, _W = 64, 768, 24, 24
_HW = _H * _W
_BB = 4
_RPW = 144
_NSTAGE = 2
_SROWS = _RPW * _NSTAGE
_BPT = 2
_DPB = _HW // _SROWS


def _mul_body(t_ref, s_ref, emb_ref, x_ref, out_ref):
    del t_ref
    m = jax.nn.sigmoid(s_ref[0, 0] * emb_ref[0, 0, :])
    out_ref[...] = x_ref[...] * m[None, None, :]


def _sc_mask_body(t_hbm, s_hbm, emb_hbm, mask_hbm, tv, sv, row, slab,
                  shared, sems):
    core = jax.lax.axis_index("c")
    sid = jax.lax.axis_index("s")

    @pl.when(sid < _NSTAGE)
    def _():
        pltpu.sync_copy(t_hbm, tv.at[pl.ds(0, 1)])
        pltpu.sync_copy(s_hbm, sv.at[pl.ds(0, 1)])
        ts = tv[...][0]
        sval = sv[...][0]
        pltpu.sync_copy(emb_hbm.at[pl.ds(ts, 1)], row)

        @pl.loop(0, _C, step=16)
        def _(j):
            z = row[0, 0, pl.ds(j, 16)]
            slab[0, pl.ds(j, 16)] = 1.0 / (1.0 + jnp.exp(-sval * z))

        @pl.loop(1, _RPW)
        def _(r):
            @pl.loop(0, _C, step=16)
            def _(j):
                slab[r, pl.ds(j, 16)] = slab[0, pl.ds(j, 16)]

        pltpu.sync_copy(slab, shared.at[pl.ds(sid * _RPW, _RPW)])

    plsc.subcore_barrier()

    base = core * 32 + sid * _BPT

    def cp(i, j):
        return pltpu.make_async_copy(
            shared,
            mask_hbm.at[base + i, pl.ds(j * _SROWS, _SROWS)],
            sems.at[i * _DPB + j])

    for i in range(_BPT):
        for j in range(_DPB):
            cp(i, j).start()
    for i in range(_BPT):
        for j in range(_DPB):
            cp(i, j).wait()


def kernel(t, x, s, embedding):
    xt = jnp.transpose(x, (0, 2, 3, 1)).reshape(_B, _HW, _C)
    s2 = s.reshape(1, 1)
    t32 = t.astype(jnp.int32)
    emb3 = embedding.reshape(100, 1, _C)

    sc_mask = pl.kernel(
        _sc_mask_body,
        out_type=jax.ShapeDtypeStruct((_B, _HW, _C), jnp.float32),
        mesh=plsc.VectorSubcoreMesh(core_axis_name="c", subcore_axis_name="s"),
        scratch_types=[
            pltpu.VMEM((16,), jnp.int32),
            pltpu.VMEM((16,), jnp.float32),
            pltpu.VMEM((1, 1, _C), jnp.float32),
            pltpu.VMEM((_RPW, _C), jnp.float32),
            pltpu.VMEM_SHARED((_SROWS, _C), jnp.float32),
            pltpu.SemaphoreType.DMA((_BPT * _DPB,)),
        ],
    )
    mask = sc_mask(t32, s, emb3)

    out = pl.pallas_call(
        _mul_body,
        grid_spec=pltpu.PrefetchScalarGridSpec(
            num_scalar_prefetch=1,
            grid=(_B // _BB,),
            in_specs=[
                pl.BlockSpec((1, 1), lambda b, t_ref: (0, 0)),
                pl.BlockSpec((1, 1, _C), lambda b, t_ref: (t_ref[0], 0, 0)),
                pl.BlockSpec((_BB, _HW, _C), lambda b, t_ref: (b, 0, 0)),
            ],
            out_specs=pl.BlockSpec((_BB, _HW, _C), lambda b, t_ref: (b, 0, 0)),
        ),
        out_shape=jax.ShapeDtypeStruct((_B, _HW, _C), jnp.float32),
    )(t32, s2, emb3, xt)

    out4 = jnp.transpose(out.reshape(_B, _H, _W, _C), (0, 3, 1, 2))
    mask4 = jnp.transpose(mask.reshape(_B, _H, _W, _C), (0, 3, 1, 2))
    return out4, mask4


# speedup vs baseline: 1.2210x; 1.2210x over previous
import jax
import jax.numpy as jnp
from jax.experimental import pallas as pl
from jax.experimental.pallas import tpu as pltpu

_B, _C, _H, _W = 64, 768, 24, 24
_HW = _H * _W
_NS = 6


def _body(t_ref, s_ref, emb_ref, x_hbm, out_hbm, mask_hbm,
          xbuf, obuf, mbuf, sin, sout, smask):
    del t_ref
    m = jax.nn.sigmoid(s_ref[0, 0] * emb_ref[0, 0, :])
    mrow = m[None, None, :]
    mbuf[...] = jnp.broadcast_to(m[None, :], (_HW, _C))

    def cp_in(b, slot):
        return pltpu.make_async_copy(x_hbm.at[b], xbuf.at[slot], sin.at[slot])

    def cp_out(b, slot):
        return pltpu.make_async_copy(obuf.at[slot], out_hbm.at[b], sout.at[slot])

    def cp_mask(b, slot):
        return pltpu.make_async_copy(mbuf, mask_hbm.at[b], smask.at[slot])

    for i in range(_NS):
        cp_in(i, i).start()
        cp_mask(i, i).start()

    def step(b, carry):
        slot = jax.lax.rem(b, _NS)

        @pl.when(b >= _NS)
        def _():
            cp_out(b - _NS, slot).wait()
            cp_mask(b - _NS, slot).wait()
            cp_mask(b, slot).start()

        cp_in(b, slot).wait()
        obuf[pl.ds(slot, 1)] = xbuf[pl.ds(slot, 1)] * mrow
        cp_out(b, slot).start()

        @pl.when(b + _NS < _B)
        def _():
            cp_in(b + _NS, slot).start()

        return carry

    jax.lax.fori_loop(0, _B, step, 0, unroll=2)

    for i in range(_B - _NS, _B):
        cp_out(i, i % _NS).wait()
        cp_mask(i, i % _NS).wait()


def kernel(t, x, s, embedding):
    xt = jnp.transpose(x, (0, 2, 3, 1)).reshape(_B, _HW, _C)
    s2 = s.reshape(1, 1)
    t32 = t.astype(jnp.int32)

    out, mask = pl.pallas_call(
        _body,
        grid_spec=pltpu.PrefetchScalarGridSpec(
            num_scalar_prefetch=1,
            grid=(1,),
            in_specs=[
                pl.BlockSpec((1, 1), lambda i, t_ref: (0, 0)),
                pl.BlockSpec((1, 1, _C), lambda i, t_ref: (t_ref[0], 0, 0)),
                pl.BlockSpec(memory_space=pl.ANY),
            ],
            out_specs=[
                pl.BlockSpec(memory_space=pl.ANY),
                pl.BlockSpec(memory_space=pl.ANY),
            ],
            scratch_shapes=[
                pltpu.VMEM((_NS, _HW, _C), jnp.float32),
                pltpu.VMEM((_NS, _HW, _C), jnp.float32),
                pltpu.VMEM((_HW, _C), jnp.float32),
                pltpu.SemaphoreType.DMA((_NS,)),
                pltpu.SemaphoreType.DMA((_NS,)),
                pltpu.SemaphoreType.DMA((_NS,)),
            ],
        ),
        out_shape=[
            jax.ShapeDtypeStruct((_B, _HW, _C), jnp.float32),
            jax.ShapeDtypeStruct((_B, _HW, _C), jnp.float32),
        ],
    )(t32, s2, embedding.reshape(100, 1, _C), xt)

    out4 = jnp.transpose(out.reshape(_B, _H, _W, _C), (0, 3, 1, 2))
    mask4 = jnp.transpose(mask.reshape(_B, _H, _W, _C), (0, 3, 1, 2))
    return out4, mask4


# speedup vs baseline: 1.2767x; 1.0456x over previous
import jax
import jax.numpy as jnp
from jax.experimental import pallas as pl
from jax.experimental.pallas import tpu as pltpu

_B, _C, _H, _W = 64, 768, 24, 24
_HW = _H * _W
_BB = 4


def _body(t_ref, s_ref, emb_ref, x_ref, out_ref, mask_ref):
    del t_ref
    m = jax.nn.sigmoid(s_ref[0, 0] * emb_ref[0, 0, :])
    mrow = m[None, None, :]
    out_ref[...] = x_ref[...] * mrow
    mask_ref[...] = jnp.broadcast_to(mrow, (_BB, _HW, _C))


def kernel(t, x, s, embedding):
    xt = jnp.transpose(x, (0, 2, 3, 1)).reshape(_B, _HW, _C)
    s2 = s.reshape(1, 1)
    t32 = t.astype(jnp.int32)

    out, mask = pl.pallas_call(
        _body,
        grid_spec=pltpu.PrefetchScalarGridSpec(
            num_scalar_prefetch=1,
            grid=(_B // _BB,),
            in_specs=[
                pl.BlockSpec((1, 1), lambda b, t_ref: (0, 0)),
                pl.BlockSpec((1, 1, _C), lambda b, t_ref: (t_ref[0], 0, 0)),
                pl.BlockSpec((_BB, _HW, _C), lambda b, t_ref: (b, 0, 0)),
            ],
            out_specs=[
                pl.BlockSpec((_BB, _HW, _C), lambda b, t_ref: (b, 0, 0)),
                pl.BlockSpec((_BB, _HW, _C), lambda b, t_ref: (b, 0, 0)),
            ],
        ),
        out_shape=[
            jax.ShapeDtypeStruct((_B, _HW, _C), jnp.float32),
            jax.ShapeDtypeStruct((_B, _HW, _C), jnp.float32),
        ],
    )(t32, s2, embedding.reshape(100, 1, _C), xt)

    out4 = jnp.transpose(out.reshape(_B, _H, _W, _C), (0, 3, 1, 2))
    mask4 = jnp.transpose(mask.reshape(_B, _H, _W, _C), (0, 3, 1, 2))
    return out4, mask4
